# baseline (speedup 1.0000x reference)
import functools

import jax
import jax.numpy as jnp
from jax import lax
from jax.experimental import pallas as pl
from jax.experimental.pallas import tpu as pltpu
from jax.experimental.pallas import tpu_sc as plsc

N_IDENT = 10000
D = 128
N_SYM = 8192
N_EXPR = 2048
MAX_TOK = 64
N_APP = 320000
N_TAB = N_EXPR * MAX_TOK

NC, NS = 2, 16
NW = NC * NS
CHUNK = 128
STEPS = -(-N_APP // (NW * CHUNK))
N_PAD = NW * CHUNK * STEPS
ACC_ROWS = N_SYM + CHUNK
ZROWS = ACC_ROWS // NS
IDENT_STEPS = N_SYM // (NW * CHUNK)


NBUF = 2


def _sc_body(expr_tab, ident_tab, occ_idx, sym_idx, ident_idx, zrows,
             a_out, part_out,
             occ_v, sym_v, iv, dbuf, acc, sem, gsem0, gsem1):
    cid = lax.axis_index("c")
    sid = lax.axis_index("s")
    wid = (1 - cid) * NS + sid

    pltpu.sync_copy(zrows, acc.at[pl.ds(sid * ZROWS, ZROWS)])

    pltpu.sync_copy(occ_idx.at[wid], occ_v)
    pltpu.sync_copy(sym_idx.at[wid], sym_v)
    pltpu.sync_copy(ident_idx.at[wid], iv)

    for r in range(IDENT_STEPS):
        pltpu.async_copy(ident_tab.at[iv.at[r]], dbuf.at[pl.ds(0, CHUNK)],
                         sem).wait()
        pltpu.sync_copy(dbuf.at[pl.ds(0, CHUNK)],
                        a_out.at[pl.ds(wid * CHUNK * IDENT_STEPS
                                       + r * CHUNK, CHUNK)])

    plsc.subcore_barrier()

    def buf(j):
        return dbuf.at[pl.ds((lax.rem(j, 2)) * CHUNK, CHUNK)]

    pltpu.async_copy(expr_tab.at[occ_v.at[0]], dbuf.at[pl.ds(0, CHUNK)],
                     gsem0)

    def step(j, _):
        pltpu.make_async_copy(expr_tab.at[occ_v.at[j]], buf(j),
                              gsem0).wait()

        @pl.when(j + 1 < STEPS)
        def _():
            pltpu.async_copy(expr_tab.at[occ_v.at[j + 1]], buf(j + 1),
                             gsem0)

        pltpu.sync_copy(buf(j), acc.at[sym_v.at[j]], add=True)
        return 0

    lax.fori_loop(0, STEPS, step, 0)

    plsc.subcore_barrier()

    rows = N_SYM // NS
    pltpu.sync_copy(acc.at[pl.ds(sid * rows, rows)],
                    part_out.at[cid, pl.ds(sid * rows, rows)])


_sc_kernel = functools.partial(
    pl.kernel,
    out_type=[
        jax.ShapeDtypeStruct((N_SYM, D), jnp.float32),
        jax.ShapeDtypeStruct((NC, N_SYM, D), jnp.float32),
    ],
    mesh=plsc.VectorSubcoreMesh(core_axis_name="c", subcore_axis_name="s"),
    scratch_types=[
        pltpu.VMEM((STEPS, CHUNK), jnp.int32),
        pltpu.VMEM((STEPS, CHUNK), jnp.int32),
        pltpu.VMEM((IDENT_STEPS, CHUNK), jnp.int32),
        pltpu.VMEM((2 * CHUNK, D), jnp.float32),
        pltpu.VMEM_SHARED((ACC_ROWS, D), jnp.float32),
        pltpu.SemaphoreType.DMA,
        pltpu.SemaphoreType.DMA,
        pltpu.SemaphoreType.DMA,
    ],
)(_sc_body)


def _tc_body(a_ref, s0_ref, s1_ref, w_ref, o_ref):
    w = w_ref[...]
    y = lax.dot_general(a_ref[...], w[:, :D], (((1,), (1,)), ((), ())),
                        preferred_element_type=jnp.float32)
    y += lax.dot_general(s0_ref[...] + s1_ref[...], w[:, D:],
                         (((1,), (1,)), ((), ())),
                         preferred_element_type=jnp.float32)
    o_ref[...] = jnp.maximum(y, 0.0)


_TC_BLK = 1024
_tc_kernel = pl.pallas_call(
    _tc_body,
    grid=(N_SYM // _TC_BLK,),
    in_specs=[
        pl.BlockSpec((_TC_BLK, D), lambda i: (i, 0)),
        pl.BlockSpec((_TC_BLK, D), lambda i: (i, 0)),
        pl.BlockSpec((_TC_BLK, D), lambda i: (i, 0)),
        pl.BlockSpec((D, 2 * D), lambda i: (0, 0)),
    ],
    out_specs=pl.BlockSpec((_TC_BLK, D), lambda i: (i, 0)),
    out_shape=jax.ShapeDtypeStruct((N_SYM, D), jnp.float32),
)


def kernel(encoded_identifiers, full_expr_encoded, W,
           symbols_identifier_indices, expr_idx, token_idx, symbol_idx):
    flat_expr = full_expr_encoded.reshape(N_TAB, D)
    occ = (MAX_TOK * expr_idx + token_idx).astype(jnp.int32)
    pad = N_PAD - N_APP
    occ = jnp.concatenate([occ, jnp.zeros((pad,), jnp.int32)])
    sym = jnp.concatenate([symbol_idx.astype(jnp.int32),
                           jnp.full((pad,), N_SYM, jnp.int32)])
    occ = occ.reshape(NW, STEPS, CHUNK)
    sym = sym.reshape(NW, STEPS, CHUNK)
    ident_idx = symbols_identifier_indices.astype(jnp.int32).reshape(
        NW, IDENT_STEPS, CHUNK)
    zrows = jnp.zeros((ZROWS, D), jnp.float32)

    a, parts = _sc_kernel(flat_expr, encoded_identifiers, occ, sym,
                          ident_idx, zrows)
    out = _tc_kernel(a, parts[0], parts[1], W)
    return out.reshape(N_SYM // 512, 512, D)


# speedup vs baseline: 1.1189x; 1.1189x over previous
import functools

import jax
import jax.numpy as jnp
from jax import lax
from jax.experimental import pallas as pl
from jax.experimental.pallas import tpu as pltpu
from jax.experimental.pallas import tpu_sc as plsc

N_IDENT = 10000
D = 128
N_SYM = 8192
N_EXPR = 2048
MAX_TOK = 64
N_APP = 320000
N_TAB = N_EXPR * MAX_TOK

NC, NS = 2, 16
NW = NC * NS
CHUNK = 128
STEPS = -(-N_APP // (NW * CHUNK))
N_PAD = NW * CHUNK * STEPS
ACC_ROWS = N_SYM + CHUNK
ZROWS = ACC_ROWS // NS
IDENT_STEPS = N_SYM // (NW * CHUNK)

NBUF = 3
RING = 8


def _sc_body(expr_tab, ident_tab, idx_hbm, ident_idx, zrows,
             a_out, part_out,
             ring, iv, dbuf, acc, sem, isem, gsem):
    cid = lax.axis_index("c")
    sid = lax.axis_index("s")
    wid = cid * NS + sid

    pltpu.sync_copy(zrows, acc.at[pl.ds(sid * ZROWS, ZROWS)])

    pltpu.sync_copy(ident_idx.at[wid], iv)
    for r in range(IDENT_STEPS):
        pltpu.async_copy(ident_tab.at[iv.at[r]], dbuf.at[pl.ds(0, CHUNK)],
                         sem).wait()
        pltpu.sync_copy(dbuf.at[pl.ds(0, CHUNK)],
                        a_out.at[pl.ds(wid * CHUNK * IDENT_STEPS
                                       + r * CHUNK, CHUNK)])

    plsc.subcore_barrier()

    K = NBUF - 1
    P = RING - 1

    def islot(j):
        return ring.at[lax.rem(j, RING)]

    def isrc(j):
        return idx_hbm.at[wid, j]

    def buf(j):
        return dbuf.at[pl.ds(lax.rem(j, NBUF) * CHUNK, CHUNK)]

    def start_g(j):
        pltpu.async_copy(expr_tab.at[islot(j).at[0]], buf(j), gsem)

    for j in range(P):
        pltpu.async_copy(isrc(j), islot(j), isem)
    for j in range(K):
        pltpu.make_async_copy(isrc(j), islot(j), isem).wait()
        start_g(j)

    def step(j, _):
        @pl.when(j + K < STEPS)
        def _():
            pltpu.make_async_copy(isrc(j + K), islot(j + K), isem).wait()

        @pl.when(j + P < STEPS)
        def _():
            pltpu.async_copy(isrc(j + P), islot(j + P), isem)

        pltpu.make_async_copy(expr_tab.at[islot(j).at[0]], buf(j),
                              gsem).wait()

        @pl.when(j + K < STEPS)
        def _():
            start_g(j + K)

        pltpu.sync_copy(buf(j), acc.at[islot(j).at[1]], add=True)
        return 0

    lax.fori_loop(0, STEPS, step, 0)

    plsc.subcore_barrier()

    rows = N_SYM // NS
    pltpu.sync_copy(acc.at[pl.ds(sid * rows, rows)],
                    part_out.at[cid, pl.ds(sid * rows, rows)])


_sc_kernel = functools.partial(
    pl.kernel,
    out_type=[
        jax.ShapeDtypeStruct((N_SYM, D), jnp.float32),
        jax.ShapeDtypeStruct((NC, N_SYM, D), jnp.float32),
    ],
    mesh=plsc.VectorSubcoreMesh(core_axis_name="c", subcore_axis_name="s"),
    scratch_types=[
        pltpu.VMEM((RING, 2, CHUNK), jnp.int32),
        pltpu.VMEM((IDENT_STEPS, CHUNK), jnp.int32),
        pltpu.VMEM((NBUF * CHUNK, D), jnp.float32),
        pltpu.VMEM_SHARED((ACC_ROWS, D), jnp.float32),
        pltpu.SemaphoreType.DMA,
        pltpu.SemaphoreType.DMA,
        pltpu.SemaphoreType.DMA,
    ],
)(_sc_body)


def _tc_body(a_ref, s0_ref, s1_ref, w_ref, o_ref):
    w = w_ref[...]
    y = lax.dot_general(a_ref[...], w[:, :D], (((1,), (1,)), ((), ())),
                        preferred_element_type=jnp.float32)
    y += lax.dot_general(s0_ref[...] + s1_ref[...], w[:, D:],
                         (((1,), (1,)), ((), ())),
                         preferred_element_type=jnp.float32)
    o_ref[...] = jnp.maximum(y, 0.0)


_TC_BLK = 1024
_tc_kernel = pl.pallas_call(
    _tc_body,
    grid=(N_SYM // _TC_BLK,),
    in_specs=[
        pl.BlockSpec((_TC_BLK, D), lambda i: (i, 0)),
        pl.BlockSpec((_TC_BLK, D), lambda i: (i, 0)),
        pl.BlockSpec((_TC_BLK, D), lambda i: (i, 0)),
        pl.BlockSpec((D, 2 * D), lambda i: (0, 0)),
    ],
    out_specs=pl.BlockSpec((_TC_BLK, D), lambda i: (i, 0)),
    out_shape=jax.ShapeDtypeStruct((N_SYM, D), jnp.float32),
)


def kernel(encoded_identifiers, full_expr_encoded, W,
           symbols_identifier_indices, expr_idx, token_idx, symbol_idx):
    flat_expr = full_expr_encoded.reshape(N_TAB, D)
    occ = (MAX_TOK * expr_idx + token_idx).astype(jnp.int32)
    pad = N_PAD - N_APP
    occ = jnp.concatenate([occ, jnp.zeros((pad,), jnp.int32)])
    sym = jnp.concatenate([symbol_idx.astype(jnp.int32),
                           jnp.full((pad,), N_SYM, jnp.int32)])
    idx = jnp.stack([occ.reshape(NW, STEPS, CHUNK),
                     sym.reshape(NW, STEPS, CHUNK)], axis=2)
    ident_idx = symbols_identifier_indices.astype(jnp.int32).reshape(
        NW, IDENT_STEPS, CHUNK)
    zrows = jnp.zeros((ZROWS, D), jnp.float32)

    a, parts = _sc_kernel(flat_expr, encoded_identifiers, idx,
                          ident_idx, zrows)
    out = _tc_kernel(a, parts[0], parts[1], W)
    return out.reshape(N_SYM // 512, 512, D)
